# baseline (speedup 1.0000x reference)
import functools
import math

import jax
import jax.numpy as jnp
from jax import lax
from jax.experimental import pallas as pl
from jax.experimental.pallas import tpu as pltpu
from jax.experimental.pallas import tpu_sc as plsc

D = 64
IDX_CHUNK = 128


def _sc_gather_pair(user_table, item_table, uid2, iid2, n_workers, chunks):
    bpw = chunks * IDX_CHUNK
    B = n_workers * bpw
    info = plsc.get_sparse_core_info()
    NC = info.num_cores

    mesh = plsc.VectorSubcoreMesh(core_axis_name="c", subcore_axis_name="s")

    @functools.partial(
        pl.kernel,
        out_type=(
            jax.ShapeDtypeStruct((B, D), jnp.float32),
            jax.ShapeDtypeStruct((B, D), jnp.float32),
        ),
        mesh=mesh,
        compiler_params=pltpu.CompilerParams(use_tc_tiling_on_sc=False),
        scratch_types=[
            pltpu.VMEM((chunks, IDX_CHUNK), jnp.int32),
            pltpu.VMEM((chunks, IDX_CHUNK), jnp.int32),
            pltpu.VMEM((bpw, D), jnp.float32),
            pltpu.VMEM((bpw, D), jnp.float32),
            pltpu.SemaphoreType.DMA,
            pltpu.SemaphoreType.DMA,
        ],
    )
    def k(ut, it, uids, iids, ue_out, ie_out, uidx_v, iidx_v, urows_v,
          irows_v, usem, isem):
        wid = lax.axis_index("s") * NC + lax.axis_index("c")
        row0 = wid * chunks
        pltpu.sync_copy(uids.at[pl.ds(row0, chunks)], uidx_v)
        pltpu.sync_copy(iids.at[pl.ds(row0, chunks)], iidx_v)
        ucopies = []
        icopies = []
        for j in range(chunks):
            ucopies.append(pltpu.async_copy(
                ut.at[uidx_v.at[j]],
                urows_v.at[pl.ds(j * IDX_CHUNK, IDX_CHUNK)], usem))
            icopies.append(pltpu.async_copy(
                it.at[iidx_v.at[j]],
                irows_v.at[pl.ds(j * IDX_CHUNK, IDX_CHUNK)], isem))
        base = wid * bpw
        for c in ucopies:
            c.wait()
        pltpu.sync_copy(urows_v, ue_out.at[pl.ds(base, bpw)])
        for c in icopies:
            c.wait()
        pltpu.sync_copy(irows_v, ie_out.at[pl.ds(base, bpw)])

    return k(user_table, item_table, uid2, iid2)


def _mlp_body(ue_ref, ie_ref, w0_ref, b0_ref, g0_ref, bt0_ref,
              w1_ref, b1_ref, g1_ref, bt1_ref,
              w2_ref, b2_ref, g2_ref, bt2_ref,
              wo_ref, bo_ref, out_ref):
    inv = 1.0 / math.sqrt(1.0 + 1e-5)
    x = (jnp.dot(ue_ref[...], w0_ref[:D, :], preferred_element_type=jnp.float32)
         + jnp.dot(ie_ref[...], w0_ref[D:, :], preferred_element_type=jnp.float32)
         + b0_ref[...])
    x = g0_ref[...] * (jnp.maximum(x, 0.0) * inv) + bt0_ref[...]
    x = jnp.dot(x, w1_ref[...], preferred_element_type=jnp.float32) + b1_ref[...]
    x = g1_ref[...] * (jnp.maximum(x, 0.0) * inv) + bt1_ref[...]
    x = jnp.dot(x, w2_ref[...], preferred_element_type=jnp.float32) + b2_ref[...]
    x = g2_ref[...] * (jnp.maximum(x, 0.0) * inv) + bt2_ref[...]
    o = jnp.dot(x, wo_ref[...], preferred_element_type=jnp.float32) + bo_ref[...]
    out_ref[...] = jax.nn.sigmoid(o)


def _mlp(ue, ie, W0, b0, g0, bt0, W1, b1, g1, bt1, W2, b2, g2, bt2, Wo, bo,
         block_m):
    B = ue.shape[0]
    grid = (B // block_m,)

    def batch_spec(cols):
        return pl.BlockSpec((block_m, cols), lambda i: (i, 0))

    def full_spec(arr):
        return pl.BlockSpec(arr.shape, lambda i: (0,) * arr.ndim)

    row = lambda v: v.reshape(1, -1)
    args = (ue, ie, W0, row(b0), row(g0), row(bt0),
            W1, row(b1), row(g1), row(bt1),
            W2, row(b2), row(g2), row(bt2),
            Wo, row(bo))
    in_specs = [batch_spec(D), batch_spec(D)] + [full_spec(a) for a in args[2:]]
    return pl.pallas_call(
        _mlp_body,
        grid=grid,
        in_specs=in_specs,
        out_specs=pl.BlockSpec((block_m, 1), lambda i: (i, 0)),
        out_shape=jax.ShapeDtypeStruct((B, 1), jnp.float32),
    )(*args)


def kernel(user_ids, item_ids, user_table, item_table,
           W0, b0, gamma0, beta0,
           W1, b1, gamma1, beta1,
           W2, b2, gamma2, beta2,
           Wo, bo):
    B = user_ids.shape[0]
    info = plsc.get_sparse_core_info()
    n_workers = info.num_cores * info.num_subcores
    chunks = B // (n_workers * IDX_CHUNK)
    uid2 = user_ids.astype(jnp.int32).reshape(n_workers * chunks, IDX_CHUNK)
    iid2 = item_ids.astype(jnp.int32).reshape(n_workers * chunks, IDX_CHUNK)
    ue, ie = _sc_gather_pair(user_table, item_table, uid2, iid2,
                             n_workers, chunks)
    out = _mlp(ue, ie, W0, b0, gamma0, beta0, W1, b1, gamma1, beta1,
               W2, b2, gamma2, beta2, Wo, bo, block_m=2048)
    return out.reshape(B)
